# baseline (speedup 1.0000x reference)
import functools

import jax
import jax.numpy as jnp
from jax import lax
from jax.experimental import pallas as pl
from jax.experimental.pallas import tpu as pltpu
from jax.experimental.pallas import tpu_sc as plsc

_D = 128
_B = 4096
_SEQ = 50
_LP = 64
_NW = 32
_BPW = _B // _NW
_RPG = 2
_GPW = _BPW // _RPG
_NBUF = 4
_NSUP = _GPW // _NBUF
_VPT = _D // 16


def _lane_sum(v):
    for sh in (8, 4, 2, 1):
        idx = jnp.arange(16, dtype=jnp.int32) ^ sh
        v = v + v.at[idx].get(mode="promise_in_bounds")
    return v


def _tec_body(idx_hbm, mask_hbm, w_hbm, out_hbm,
              idx_v, mask_v, b0, b1, b2, b3, out_v, w0_v,
              s0, s1, s2, s3):
    bufs = (b0, b1, b2, b3)
    sems = (s0, s1, s2, s3)
    wid = lax.axis_index("s") * 2 + lax.axis_index("c")

    pltpu.sync_copy(idx_hbm.at[pl.ds(wid * _GPW, _GPW), :], idx_v)
    pltpu.sync_copy(mask_hbm.at[pl.ds(wid * (_BPW * _LP), _BPW * _LP)], mask_v)
    pltpu.sync_copy(w_hbm.at[0], w0_v)

    def mask_pass(g, _):
        for u in range(_RPG * _LP // 16):
            idx_v[g, pl.ds(16 * u, 16)] = (
                idx_v[g, pl.ds(16 * u, 16)]
                * mask_v[pl.ds(g * (_RPG * _LP) + 16 * u, 16)])
        return 0
    lax.fori_loop(0, _GPW, mask_pass, 0)

    def gather(grp, i):
        pltpu.async_copy(w_hbm.at[idx_v.at[grp]], bufs[i], sems[i])

    for i in range(_NBUF):
        gather(i, i)

    def supergroup(sg, _):
        for i in range(_NBUF):
            grp = sg * _NBUF + i
            pltpu.make_async_copy(
                w_hbm.at[idx_v.at[grp]], bufs[i], sems[i]).wait()

            accs = []
            for h in range(_RPG):
                def tok(l, acc, i=i, h=h):
                    return tuple(acc[j] + bufs[i][h * _LP + l, pl.ds(16 * j, 16)]
                                 for j in range(_VPT))
                accs.append(lax.fori_loop(
                    0, _SEQ, tok,
                    tuple(jnp.zeros((16,), jnp.float32) for _ in range(_VPT))))

            @pl.when(sg + 1 < _NSUP)
            def _(grp=grp, i=i):
                gather(grp + _NBUF, i)

            for h in range(_RPG):
                row = grp * _RPG + h
                moff = row * _LP
                msum = sum(mask_v[pl.ds(moff + 16 * k, 16)]
                           for k in range(_LP // 16))
                zcnt = sum(
                    jnp.where(idx_v[grp, pl.ds(h * _LP + 16 * k, 16)] == 0, 1, 0)
                    for k in range(_LP // 16))
                len_v = jnp.maximum(_lane_sum(msum.astype(jnp.float32)), 1.0)
                c0_v = _lane_sum(zcnt.astype(jnp.float32)) - float(_LP - _SEQ)
                inv_v = 1.0 / len_v
                for j in range(_VPT):
                    out_v[pl.ds(row * _D + 16 * j, 16)] = (
                        (accs[h][j] - c0_v * w0_v[pl.ds(16 * j, 16)]) * inv_v)
        return 0

    lax.fori_loop(0, _NSUP, supergroup, 0)
    pltpu.sync_copy(out_v, out_hbm.at[pl.ds(wid * (_BPW * _D), _BPW * _D)])


_mesh = plsc.VectorSubcoreMesh(core_axis_name="c", subcore_axis_name="s")

_encode = functools.partial(
    pl.kernel,
    out_type=jax.ShapeDtypeStruct((_B * _D,), jnp.float32),
    mesh=_mesh,
    scratch_types=[
        pltpu.VMEM((_NW * _GPW // _NW, _RPG * _LP), jnp.int32),
        pltpu.VMEM((_BPW * _LP,), jnp.int32),
        pltpu.VMEM((_RPG * _LP, _D), jnp.float32),
        pltpu.VMEM((_RPG * _LP, _D), jnp.float32),
        pltpu.VMEM((_RPG * _LP, _D), jnp.float32),
        pltpu.VMEM((_RPG * _LP, _D), jnp.float32),
        pltpu.VMEM((_BPW * _D,), jnp.float32),
        pltpu.VMEM((_D,), jnp.float32),
        pltpu.SemaphoreType.DMA,
        pltpu.SemaphoreType.DMA,
        pltpu.SemaphoreType.DMA,
        pltpu.SemaphoreType.DMA,
    ],
)(_tec_body)


@jax.jit
def kernel(input_ids, attention_mask, W):
    pad = ((0, 0), (0, _LP - _SEQ))
    idxp = jnp.pad(input_ids, pad).reshape(-1, _RPG * _LP)
    maskp = jnp.pad(attention_mask, pad).reshape(-1)
    out = _encode(idxp, maskp, W)
    return out.reshape(_B, _D)


# speedup vs baseline: 85.6127x; 85.6127x over previous
import functools

import jax
import jax.numpy as jnp
from jax import lax
from jax.experimental import pallas as pl
from jax.experimental.pallas import tpu as pltpu
from jax.experimental.pallas import tpu_sc as plsc

_D = 128
_B = 4096
_SEQ = 50
_LP = 64
_NW = 32
_BPW = _B // _NW
_NBUF = 4
_NGRP = _BPW // _NBUF
_VPT = _D // 16
_WPW = _BPW * _LP


def _lane_sum(v):
    for sh in (8, 4, 2, 1):
        idx = jnp.arange(16, dtype=jnp.int32) ^ sh
        v = v + v.at[idx].get(mode="promise_in_bounds")
    return v


def _tec_body(idx_hbm, mask_hbm, w_hbm, out_hbm,
              idx_v, mask_v, wgt_v, b0, b1, b2, b3, out_v,
              s0, s1, s2, s3):
    bufs = (b0, b1, b2, b3)
    sems = (s0, s1, s2, s3)
    wid = lax.axis_index("s") * 2 + lax.axis_index("c")
    base = wid * _WPW

    pltpu.sync_copy(idx_hbm.at[pl.ds(base, _WPW)], idx_v)
    pltpu.sync_copy(mask_hbm.at[pl.ds(base, _WPW)], mask_v)

    def weight_pass(r, _):
        moff = r * _LP
        mvecs = [mask_v[pl.ds(moff + 16 * k, 16)] for k in range(_LP // 16)]
        msum = mvecs[0] + mvecs[1] + mvecs[2] + mvecs[3]
        len_v = jnp.maximum(_lane_sum(msum.astype(jnp.float32)), 1.0)
        inv_v = 1.0 / len_v
        for k in range(_LP // 16):
            ivec = idx_v[pl.ds(moff + 16 * k, 16)]
            w = jnp.where((ivec != 0) & (mvecs[k] != 0), inv_v, 0.0)
            wgt_v[pl.ds(moff + 16 * k, 16)] = w
        return 0
    lax.fori_loop(0, _BPW, weight_pass, 0)

    def gather(row, i):
        pltpu.async_copy(
            w_hbm.at[idx_v.at[pl.ds(row * _LP, _SEQ)]], bufs[i], sems[i])

    for i in range(_NBUF):
        gather(i, i)

    def group(g, _):
        for i in range(_NBUF):
            row = g * _NBUF + i
            pltpu.make_async_copy(
                w_hbm.at[idx_v.at[pl.ds(row * _LP, _SEQ)]], bufs[i], sems[i]
            ).wait()

            def tok(l, acc, i=i, row=row):
                wv = wgt_v[pl.ds(row * _LP + ((l >> 4) << 4), 16)]
                wb = wv.at[jnp.zeros((16,), jnp.int32) + (l & 15)].get(
                    mode="promise_in_bounds")
                return tuple(acc[j] + wb * bufs[i][l, pl.ds(16 * j, 16)]
                             for j in range(_VPT))
            acc = lax.fori_loop(
                0, _SEQ, tok,
                tuple(jnp.zeros((16,), jnp.float32) for _ in range(_VPT)))

            @pl.when(g + 1 < _NGRP)
            def _(row=row, i=i):
                gather(row + _NBUF, i)

            for j in range(_VPT):
                out_v[pl.ds(row * _D + 16 * j, 16)] = acc[j]
        return 0

    lax.fori_loop(0, _NGRP, group, 0)
    pltpu.sync_copy(out_v, out_hbm.at[pl.ds(wid * (_BPW * _D), _BPW * _D)])


_mesh = plsc.VectorSubcoreMesh(core_axis_name="c", subcore_axis_name="s")

_encode = functools.partial(
    pl.kernel,
    out_type=jax.ShapeDtypeStruct((_B * _D,), jnp.float32),
    mesh=_mesh,
    scratch_types=[
        pltpu.VMEM((_WPW,), jnp.int32),
        pltpu.VMEM((_WPW,), jnp.int32),
        pltpu.VMEM((_WPW,), jnp.float32),
        pltpu.VMEM((_SEQ, _D), jnp.float32),
        pltpu.VMEM((_SEQ, _D), jnp.float32),
        pltpu.VMEM((_SEQ, _D), jnp.float32),
        pltpu.VMEM((_SEQ, _D), jnp.float32),
        pltpu.VMEM((_BPW * _D,), jnp.float32),
        pltpu.SemaphoreType.DMA,
        pltpu.SemaphoreType.DMA,
        pltpu.SemaphoreType.DMA,
        pltpu.SemaphoreType.DMA,
    ],
)(_tec_body)


@jax.jit
def kernel(input_ids, attention_mask, W):
    pad = ((0, 0), (0, _LP - _SEQ))
    idxp = jnp.pad(input_ids, pad).reshape(-1)
    maskp = jnp.pad(attention_mask, pad).reshape(-1)
    out = _encode(idxp, maskp, W)
    return out.reshape(_B, _D)


# speedup vs baseline: 97.5406x; 1.1393x over previous
import functools

import jax
import jax.numpy as jnp
from jax import lax
from jax.experimental import pallas as pl
from jax.experimental.pallas import tpu as pltpu
from jax.experimental.pallas import tpu_sc as plsc

_D = 128
_B = 4096
_SEQ = 50
_LP = 64
_NW = 32
_BPW = _B // _NW
_NBUF = 8
_NGRP = _BPW // _NBUF
_VPT = _D // 16
_WPW = _BPW * _LP


def _lane_sum(v):
    for sh in (8, 4, 2, 1):
        idx = jnp.arange(16, dtype=jnp.int32) ^ sh
        v = v + v.at[idx].get(mode="promise_in_bounds")
    return v


def _tec_body(idx_hbm, mask_hbm, w_hbm, out_hbm,
              idx_v, mask_v, wgt_v, b0, b1, b2, b3, b4, b5, b6, b7, out_v,
              s0, s1, s2, s3, s4, s5, s6, s7):
    bufs = (b0, b1, b2, b3, b4, b5, b6, b7)
    sems = (s0, s1, s2, s3, s4, s5, s6, s7)
    wid = lax.axis_index("s") * 2 + lax.axis_index("c")
    base = wid * _WPW

    pltpu.sync_copy(idx_hbm.at[pl.ds(base, _WPW)], idx_v)
    pltpu.sync_copy(mask_hbm.at[pl.ds(base, _WPW)], mask_v)

    def weight_pass(r, _):
        moff = r * _LP
        mvecs = [mask_v[pl.ds(moff + 16 * k, 16)] for k in range(_LP // 16)]
        msum = mvecs[0] + mvecs[1] + mvecs[2] + mvecs[3]
        len_v = jnp.maximum(_lane_sum(msum.astype(jnp.float32)), 1.0)
        inv_v = 1.0 / len_v
        for k in range(_LP // 16):
            ivec = idx_v[pl.ds(moff + 16 * k, 16)]
            w = jnp.where((ivec != 0) & (mvecs[k] != 0), inv_v, 0.0)
            wgt_v[pl.ds(moff + 16 * k, 16)] = w
        return 0
    lax.fori_loop(0, _BPW, weight_pass, 0)

    def gather(row, i):
        pltpu.async_copy(
            w_hbm.at[idx_v.at[pl.ds(row * _LP, _SEQ)]], bufs[i], sems[i])

    for i in range(_NBUF):
        gather(i, i)

    def group(g, _):
        for i in range(_NBUF):
            row = g * _NBUF + i
            pltpu.make_async_copy(
                w_hbm.at[idx_v.at[pl.ds(row * _LP, _SEQ)]], bufs[i], sems[i]
            ).wait()

            acc = tuple(jnp.zeros((16,), jnp.float32) for _ in range(_VPT))
            for k in range(_SEQ // 16 + 1):
                wv = wgt_v[pl.ds(row * _LP + 16 * k, 16)]

                def tok(u, a, i=i, k=k, wv=wv):
                    wb = wv.at[jnp.zeros((16,), jnp.int32) + u].get(
                        mode="promise_in_bounds")
                    return tuple(
                        a[j] + wb * bufs[i][16 * k + u, pl.ds(16 * j, 16)]
                        for j in range(_VPT))
                acc = lax.fori_loop(
                    0, min(16, _SEQ - 16 * k), tok, acc)

            @pl.when(g + 1 < _NGRP)
            def _(row=row, i=i):
                gather(row + _NBUF, i)

            for j in range(_VPT):
                out_v[pl.ds(row * _D + 16 * j, 16)] = acc[j]
        return 0

    lax.fori_loop(0, _NGRP, group, 0)
    pltpu.sync_copy(out_v, out_hbm.at[pl.ds(wid * (_BPW * _D), _BPW * _D)])


_mesh = plsc.VectorSubcoreMesh(core_axis_name="c", subcore_axis_name="s")

_encode = functools.partial(
    pl.kernel,
    out_type=jax.ShapeDtypeStruct((_B * _D,), jnp.float32),
    mesh=_mesh,
    scratch_types=[
        pltpu.VMEM((_WPW,), jnp.int32),
        pltpu.VMEM((_WPW,), jnp.int32),
        pltpu.VMEM((_WPW,), jnp.float32),
        pltpu.VMEM((_SEQ, _D), jnp.float32),
        pltpu.VMEM((_SEQ, _D), jnp.float32),
        pltpu.VMEM((_SEQ, _D), jnp.float32),
        pltpu.VMEM((_SEQ, _D), jnp.float32),
        pltpu.VMEM((_SEQ, _D), jnp.float32),
        pltpu.VMEM((_SEQ, _D), jnp.float32),
        pltpu.VMEM((_SEQ, _D), jnp.float32),
        pltpu.VMEM((_SEQ, _D), jnp.float32),
        pltpu.VMEM((_BPW * _D,), jnp.float32),
        pltpu.SemaphoreType.DMA,
        pltpu.SemaphoreType.DMA,
        pltpu.SemaphoreType.DMA,
        pltpu.SemaphoreType.DMA,
        pltpu.SemaphoreType.DMA,
        pltpu.SemaphoreType.DMA,
        pltpu.SemaphoreType.DMA,
        pltpu.SemaphoreType.DMA,
    ],
)(_tec_body)


@jax.jit
def kernel(input_ids, attention_mask, W):
    pad = ((0, 0), (0, _LP - _SEQ))
    idxp = jnp.pad(input_ids, pad).reshape(-1)
    maskp = jnp.pad(attention_mask, pad).reshape(-1)
    out = _encode(idxp, maskp, W)
    return out.reshape(_B, _D)


# speedup vs baseline: 98.9609x; 1.0146x over previous
import functools

import jax
import jax.numpy as jnp
from jax import lax
from jax.experimental import pallas as pl
from jax.experimental.pallas import tpu as pltpu
from jax.experimental.pallas import tpu_sc as plsc

_D = 128
_B = 4096
_SEQ = 50
_LP = 64
_NW = 32
_BPW = _B // _NW
_NBUF = 8
_NGRP = _BPW // _NBUF
_VPT = _D // 16
_WPW = _BPW * _LP


def _lane_sum(v):
    for sh in (8, 4, 2, 1):
        idx = jnp.arange(16, dtype=jnp.int32) ^ sh
        v = v + v.at[idx].get(mode="promise_in_bounds")
    return v


def _tec_body(idx_hbm, mask_hbm, w_hbm, out_hbm,
              idx_v, mask_v, wgt_v, b0, b1, b2, b3, b4, b5, b6, b7, out_v,
              s0, s1, s2, s3, s4, s5, s6, s7):
    bufs = (b0, b1, b2, b3, b4, b5, b6, b7)
    sems = (s0, s1, s2, s3, s4, s5, s6, s7)
    wid = lax.axis_index("s") * 2 + lax.axis_index("c")
    base = wid * _WPW

    pltpu.sync_copy(idx_hbm.at[pl.ds(base, _WPW)], idx_v)
    pltpu.sync_copy(mask_hbm.at[pl.ds(base, _WPW)], mask_v)

    def gather(row, i):
        pltpu.async_copy(
            w_hbm.at[idx_v.at[pl.ds(row * _LP, _SEQ)]], bufs[i], sems[i])

    for i in range(_NBUF):
        gather(i, i)

    def weight_pass(r, _):
        moff = r * _LP
        mvecs = [mask_v[pl.ds(moff + 16 * k, 16)] for k in range(_LP // 16)]
        msum = mvecs[0] + mvecs[1] + mvecs[2] + mvecs[3]
        len_v = jnp.maximum(_lane_sum(msum.astype(jnp.float32)), 1.0)
        inv_v = 1.0 / len_v
        for k in range(_LP // 16):
            ivec = idx_v[pl.ds(moff + 16 * k, 16)]
            w = jnp.where((ivec != 0) & (mvecs[k] != 0), inv_v, 0.0)
            wgt_v[pl.ds(moff + 16 * k, 16)] = w
        return 0
    lax.fori_loop(0, _BPW, weight_pass, 0)

    def group(g, _):
        for i in range(_NBUF):
            row = g * _NBUF + i
            pltpu.make_async_copy(
                w_hbm.at[idx_v.at[pl.ds(row * _LP, _SEQ)]], bufs[i], sems[i]
            ).wait()

            acc = tuple(jnp.zeros((16,), jnp.float32) for _ in range(_VPT))
            for k in range(_SEQ // 16 + 1):
                wv = wgt_v[pl.ds(row * _LP + 16 * k, 16)]

                def tok(u, a, i=i, k=k, wv=wv):
                    wb = wv.at[jnp.zeros((16,), jnp.int32) + u].get(
                        mode="promise_in_bounds")
                    return tuple(
                        a[j] + wb * bufs[i][16 * k + u, pl.ds(16 * j, 16)]
                        for j in range(_VPT))
                acc = lax.fori_loop(
                    0, min(16, _SEQ - 16 * k), tok, acc)

            @pl.when(g + 1 < _NGRP)
            def _(row=row, i=i):
                gather(row + _NBUF, i)

            for j in range(_VPT):
                out_v[pl.ds(row * _D + 16 * j, 16)] = acc[j]
        return 0

    lax.fori_loop(0, _NGRP, group, 0)
    pltpu.sync_copy(out_v, out_hbm.at[pl.ds(wid * (_BPW * _D), _BPW * _D)])


_mesh = plsc.VectorSubcoreMesh(core_axis_name="c", subcore_axis_name="s")

_encode = functools.partial(
    pl.kernel,
    out_type=jax.ShapeDtypeStruct((_B * _D,), jnp.float32),
    mesh=_mesh,
    scratch_types=[
        pltpu.VMEM((_WPW,), jnp.int32),
        pltpu.VMEM((_WPW,), jnp.int32),
        pltpu.VMEM((_WPW,), jnp.float32),
        pltpu.VMEM((_SEQ, _D), jnp.float32),
        pltpu.VMEM((_SEQ, _D), jnp.float32),
        pltpu.VMEM((_SEQ, _D), jnp.float32),
        pltpu.VMEM((_SEQ, _D), jnp.float32),
        pltpu.VMEM((_SEQ, _D), jnp.float32),
        pltpu.VMEM((_SEQ, _D), jnp.float32),
        pltpu.VMEM((_SEQ, _D), jnp.float32),
        pltpu.VMEM((_SEQ, _D), jnp.float32),
        pltpu.VMEM((_BPW * _D,), jnp.float32),
        pltpu.SemaphoreType.DMA,
        pltpu.SemaphoreType.DMA,
        pltpu.SemaphoreType.DMA,
        pltpu.SemaphoreType.DMA,
        pltpu.SemaphoreType.DMA,
        pltpu.SemaphoreType.DMA,
        pltpu.SemaphoreType.DMA,
        pltpu.SemaphoreType.DMA,
    ],
)(_tec_body)


@jax.jit
def kernel(input_ids, attention_mask, W):
    pad = ((0, 0), (0, _LP - _SEQ))
    idxp = jnp.pad(input_ids, pad).reshape(-1)
    maskp = jnp.pad(attention_mask, pad).reshape(-1)
    out = _encode(idxp, maskp, W)
    return out.reshape(_B, _D)
